# Initial kernel scaffold; baseline (speedup 1.0000x reference)
import functools

import jax
import jax.numpy as jnp
from jax import lax
from jax.experimental import pallas as pl
from jax.experimental.pallas import tpu as pltpu
from jax.experimental.pallas import tpu_sc as plsc

_NC = 2
_NS = 16
_NW = _NC * _NS
_LANES = 16

_GCHUNK = 128


def _pool_kernel_body(B, L, E, CB, x_hbm, table_hbm, pooled_hbm,
                      idx_v, rows_v, pooled_v, sem):
  b_per_w = B // _NW
  n_chunks = b_per_w // CB
  n_idx = CB * L
  sizes = []
  off = 0
  while off < n_idx:
    sizes.append(min(_GCHUNK, n_idx - off))
    off += _GCHUNK

  wid = lax.axis_index("s") * _NC + lax.axis_index("c")
  base_row = wid * b_per_w

  def chunk_body(c, carry):
    flat_base = (base_row + c * CB) * L
    pltpu.sync_copy(x_hbm.at[pl.ds(flat_base, n_idx)], idx_v)
    copies = []
    o = 0
    for sz in sizes:
      copies.append(
          pltpu.async_copy(
              table_hbm.at[idx_v.at[pl.ds(o, sz)]],
              rows_v.at[pl.ds(o, sz)],
              sem,
          ))
      o += sz
    for cp in copies:
      cp.wait()

    for e in range(CB):
      row0 = e * L

      def acc_body(i, acc):
        a0, a1 = acc
        r = row0 + i
        return (a0 + rows_v[r, pl.ds(0, _LANES)],
                a1 + rows_v[r, pl.ds(_LANES, _LANES)])

      a0, a1 = lax.fori_loop(
          0, L, acc_body,
          (jnp.zeros((_LANES,), jnp.float32),
           jnp.zeros((_LANES,), jnp.float32)))
      pout = c * CB + e
      pooled_v[pout, pl.ds(0, _LANES)] = a0
      pooled_v[pout, pl.ds(_LANES, _LANES)] = a1
    return carry

  lax.fori_loop(0, n_chunks, chunk_body, 0)
  pltpu.sync_copy(pooled_v, pooled_hbm.at[pl.ds(base_row, b_per_w)])


def _make_pool_kernel(B, L, E, CB=8):
  b_per_w = B // _NW
  mesh = plsc.VectorSubcoreMesh(
      core_axis_name="c", subcore_axis_name="s",
      num_cores=_NC, num_subcores=_NS)
  return pl.kernel(
      functools.partial(_pool_kernel_body, B, L, E, CB),
      out_type=jax.ShapeDtypeStruct((B, E), jnp.float32),
      mesh=mesh,
      scratch_types=[
          pltpu.VMEM((CB * L,), jnp.int32),
          pltpu.VMEM((CB * L, E), jnp.float32),
          pltpu.VMEM((b_per_w, E), jnp.float32),
          pltpu.SemaphoreType.DMA,
      ],
      name="sc_embed_pool",
  )


def _mlp_body(inv_l, p_ref, w1_ref, b1_ref, w2_ref, b2_ref, o_ref):
  p = p_ref[...] * inv_l
  h = jnp.maximum(
      jnp.dot(p, w1_ref[...], preferred_element_type=jnp.float32)
      + b1_ref[...], 0.0)
  o_ref[...] = (
      jnp.dot(h, w2_ref[...], preferred_element_type=jnp.float32)
      + b2_ref[...])


def kernel(x, table, W1, b1, W2, b2):
  B, L = x.shape
  H = W1.shape[1]
  O = W2.shape[1]
  E = table.shape[1]

  pooled = _make_pool_kernel(B, L, E)(
      x.reshape(-1).astype(jnp.int32), table)

  mlp = pl.pallas_call(
      functools.partial(_mlp_body, 1.0 / L),
      out_shape=jax.ShapeDtypeStruct((B, O), jnp.float32),
  )
  return mlp(pooled, W1, b1.reshape(1, H), W2, b2.reshape(1, O))


# speedup vs baseline: 10.6638x; 10.6638x over previous
import functools

import jax
import jax.numpy as jnp
from jax import lax
from jax.experimental import pallas as pl
from jax.experimental.pallas import tpu as pltpu
from jax.experimental.pallas import tpu_sc as plsc

_NC = 2
_NS = 16
_NW = _NC * _NS
_LANES = 16

_GCHUNK = 128


def _pool_kernel_body(B, L, E, CB, x_hbm, table_hbm, pooled_hbm,
                      idx_v, rows_v, pooled_v, sem):
  b_per_w = B // _NW
  n_chunks = b_per_w // CB
  n_idx = CB * L
  sizes = []
  off = 0
  while off < n_idx:
    sizes.append(min(_GCHUNK, n_idx - off))
    off += _GCHUNK

  wid = lax.axis_index("s") * _NC + lax.axis_index("c")
  base_row = wid * b_per_w

  def chunk_body(c, carry):
    flat_base = (base_row + c * CB) * L
    pltpu.sync_copy(x_hbm.at[pl.ds(flat_base, n_idx)], idx_v)
    copies = []
    o = 0
    for sz in sizes:
      copies.append(
          pltpu.async_copy(
              table_hbm.at[idx_v.at[pl.ds(o, sz)]],
              rows_v.at[pl.ds(o, sz)],
              sem,
          ))
      o += sz
    for cp in copies:
      cp.wait()

    for e in range(CB):
      row0 = e * L

      def acc_body(i, acc):
        a0, a1 = acc
        r = row0 + i
        return (a0 + rows_v[r, pl.ds(0, _LANES)],
                a1 + rows_v[r, pl.ds(_LANES, _LANES)])

      a0, a1 = lax.fori_loop(
          0, L, acc_body,
          (jnp.zeros((_LANES,), jnp.float32),
           jnp.zeros((_LANES,), jnp.float32)))
      pout = c * CB + e
      pooled_v[pout, pl.ds(0, _LANES)] = a0
      pooled_v[pout, pl.ds(_LANES, _LANES)] = a1
    return carry

  lax.fori_loop(0, n_chunks, chunk_body, 0)
  pltpu.sync_copy(pooled_v, pooled_hbm.at[pl.ds(base_row, b_per_w)])


def _make_pool_kernel(B, L, E, CB=8):
  b_per_w = B // _NW
  mesh = plsc.VectorSubcoreMesh(
      core_axis_name="c", subcore_axis_name="s",
      num_cores=_NC, num_subcores=_NS)
  return pl.kernel(
      functools.partial(_pool_kernel_body, B, L, E, CB),
      out_type=jax.ShapeDtypeStruct((B, E), jnp.float32),
      mesh=mesh,
      scratch_types=[
          pltpu.VMEM((CB * L,), jnp.int32),
          pltpu.VMEM((CB * L, E), jnp.float32),
          pltpu.VMEM((b_per_w, E), jnp.float32),
          pltpu.SemaphoreType.DMA,
      ],
      compiler_params=pltpu.CompilerParams(use_tc_tiling_on_sc=False),
      name="sc_embed_pool",
  )


def _mlp_body(inv_l, p_ref, w1_ref, b1_ref, w2_ref, b2_ref, o_ref):
  p = p_ref[...] * inv_l
  h = jnp.maximum(
      jnp.dot(p, w1_ref[...], preferred_element_type=jnp.float32)
      + b1_ref[...], 0.0)
  o_ref[...] = (
      jnp.dot(h, w2_ref[...], preferred_element_type=jnp.float32)
      + b2_ref[...])


def kernel(x, table, W1, b1, W2, b2):
  B, L = x.shape
  H = W1.shape[1]
  O = W2.shape[1]
  E = table.shape[1]

  pooled = _make_pool_kernel(B, L, E)(
      x.reshape(-1).astype(jnp.int32), table)

  mlp = pl.pallas_call(
      functools.partial(_mlp_body, 1.0 / L),
      out_shape=jax.ShapeDtypeStruct((B, O), jnp.float32),
  )
  return mlp(pooled, W1, b1.reshape(1, H), W2, b2.reshape(1, O))


# speedup vs baseline: 16.2197x; 1.5210x over previous
import functools

import jax
import jax.numpy as jnp
from jax import lax
from jax.experimental import pallas as pl
from jax.experimental.pallas import tpu as pltpu
from jax.experimental.pallas import tpu_sc as plsc

_NC = 2
_NS = 16
_NW = _NC * _NS
_LANES = 16

_GCHUNK = 128


def _pool_kernel_body(B, L, E, CB, U, x_hbm, table_hbm, pooled_hbm,
                      idx_v, rows_v, pooled_v, sem0, sem1):
  b_per_w = B // _NW
  n_chunks = b_per_w // CB
  n_idx = CB * L
  sizes = []
  off = 0
  while off < n_idx:
    sizes.append(min(_GCHUNK, n_idx - off))
    off += _GCHUNK
  sems = (sem0, sem1)

  wid = lax.axis_index("s") * _NC + lax.axis_index("c")
  base_row = wid * b_per_w
  flat0 = base_row * L

  def load_idx(c, buf):
    pltpu.sync_copy(x_hbm.at[pl.ds(flat0 + c * n_idx, n_idx)],
                    idx_v.at[buf])

  def fire(buf):
    o = 0
    for sz in sizes:
      pltpu.async_copy(
          table_hbm.at[idx_v.at[buf, pl.ds(o, sz)]],
          rows_v.at[buf, pl.ds(o, sz)],
          sems[buf],
      )
      o += sz

  def wait(buf):
    pltpu.make_async_copy(
        table_hbm.at[pl.ds(0, n_idx)], rows_v.at[buf], sems[buf]).wait()

  def accumulate(buf, c):
    for e in range(CB):
      row0 = e * L

      def acc_body(i, acc):
        a0, a1, c0, c1 = acc
        r = row0 + i * U
        for u in range(0, U, 2):
          a0 = a0 + rows_v[buf, r + u, pl.ds(0, _LANES)]
          a1 = a1 + rows_v[buf, r + u, pl.ds(_LANES, _LANES)]
          c0 = c0 + rows_v[buf, r + u + 1, pl.ds(0, _LANES)]
          c1 = c1 + rows_v[buf, r + u + 1, pl.ds(_LANES, _LANES)]
        return a0, a1, c0, c1

      z = jnp.zeros((_LANES,), jnp.float32)
      a0, a1, c0, c1 = lax.fori_loop(0, L // U, acc_body, (z, z, z, z))
      pout = c * CB + e
      pooled_v[pout, pl.ds(0, _LANES)] = a0 + c0
      pooled_v[pout, pl.ds(_LANES, _LANES)] = a1 + c1

  n2 = n_chunks // 2
  load_idx(0, 0)
  fire(0)

  def body2(i, carry):
    load_idx(2 * i + 1, 1)
    fire(1)
    wait(0)
    accumulate(0, 2 * i)

    @pl.when(i < n2 - 1)
    def _():
      load_idx(2 * i + 2, 0)
      fire(0)

    wait(1)
    accumulate(1, 2 * i + 1)
    return carry

  lax.fori_loop(0, n2, body2, 0)
  pltpu.sync_copy(pooled_v, pooled_hbm.at[pl.ds(base_row, b_per_w)])


def _make_pool_kernel(B, L, E, CB=8, U=8):
  b_per_w = B // _NW
  mesh = plsc.VectorSubcoreMesh(
      core_axis_name="c", subcore_axis_name="s",
      num_cores=_NC, num_subcores=_NS)
  return pl.kernel(
      functools.partial(_pool_kernel_body, B, L, E, CB, U),
      out_type=jax.ShapeDtypeStruct((B, E), jnp.float32),
      mesh=mesh,
      scratch_types=[
          pltpu.VMEM((2, CB * L), jnp.int32),
          pltpu.VMEM((2, CB * L, E), jnp.float32),
          pltpu.VMEM((b_per_w, E), jnp.float32),
          pltpu.SemaphoreType.DMA,
          pltpu.SemaphoreType.DMA,
      ],
      compiler_params=pltpu.CompilerParams(use_tc_tiling_on_sc=False),
      name="sc_embed_pool",
  )


def _mlp_body(inv_l, p_ref, w1_ref, b1_ref, w2_ref, b2_ref, o_ref):
  p = p_ref[...] * inv_l
  h = jnp.maximum(
      jnp.dot(p, w1_ref[...], preferred_element_type=jnp.float32)
      + b1_ref[...], 0.0)
  o_ref[...] = (
      jnp.dot(h, w2_ref[...], preferred_element_type=jnp.float32)
      + b2_ref[...])


def kernel(x, table, W1, b1, W2, b2):
  B, L = x.shape
  H = W1.shape[1]
  O = W2.shape[1]
  E = table.shape[1]

  pooled = _make_pool_kernel(B, L, E)(
      x.reshape(-1).astype(jnp.int32), table)

  mlp = pl.pallas_call(
      functools.partial(_mlp_body, 1.0 / L),
      out_shape=jax.ShapeDtypeStruct((B, O), jnp.float32),
  )
  return mlp(pooled, W1, b1.reshape(1, H), W2, b2.reshape(1, O))
